# Initial kernel scaffold; baseline (speedup 1.0000x reference)
import jax
import jax.numpy as jnp
from jax.experimental import pallas as pl


def kernel(logits, k, p):
    raise NotImplementedError("write your pallas kernel here")


# speedup vs baseline: 49.3417x; 49.3417x over previous
import functools

import jax
import jax.numpy as jnp
from jax import lax
from jax.experimental import pallas as pl
from jax.experimental.pallas import tpu as pltpu

_LANE = 128
_INT_MIN = -2147483648
_MASK31 = 0x7FFFFFFF
_NEG_INF = float("-inf")


def _body(v_ref, k_ref, p_ref, o_ref, skey_ref, e_ref):
    v = v_ref[...]
    b = lax.bitcast_convert_type(v, jnp.int32)
    skey = jnp.where(b >= 0, b, b ^ _MASK31)
    skey_ref[...] = skey
    m = jnp.max(v, axis=1, keepdims=True)
    maxkey = jnp.max(skey, axis=1, keepdims=True)
    e = jnp.exp(v - m)
    e_ref[...] = e
    kv = k_ref[:, :1]
    pv = p_ref[:, :1]

    def cnt_ge(cand):
        hit = (skey_ref[...] >= cand).astype(jnp.int32)
        return jnp.sum(hit, axis=1, keepdims=True)

    zero = jnp.zeros_like(kv)
    base = jnp.where(cnt_ge(zero) >= kv, 0, _INT_MIN)

    def step1(i, rem):
        bit = jnp.left_shift(jnp.int32(1), 30 - i)
        cand = base + (rem | bit)
        return jnp.where(cnt_ge(cand) >= kv, rem | bit, rem)

    tk = base + lax.fori_loop(0, 31, step1, zero)

    s = jnp.sum(jnp.where(skey_ref[...] >= tk, e_ref[...], 0.0),
                axis=1, keepdims=True)
    ps = pv * s

    def mass_gt(cand):
        w = jnp.where(skey_ref[...] > cand, e_ref[...], 0.0)
        return jnp.sum(w, axis=1, keepdims=True)

    base2 = jnp.where(mass_gt(zero) >= ps, 0, _INT_MIN)

    def step2(i, rem):
        bit = jnp.left_shift(jnp.int32(1), 30 - i)
        cand = base2 + (rem | bit)
        return jnp.where(mass_gt(cand) >= ps, rem | bit, rem)

    mp = base2 + lax.fori_loop(0, 31, step2, zero)

    sk = skey_ref[...]
    keep = (sk >= tk) & ((sk > mp) | (sk == maxkey))
    o_ref[...] = jnp.where(keep, v, _NEG_INF)


@functools.partial(jax.jit, static_argnames=())
def kernel(logits, k, p):
    bsz, vocab = logits.shape
    vp = pl.cdiv(vocab, _LANE) * _LANE
    rblk = 8
    logits = logits.astype(jnp.float32)
    if vp != vocab:
        pad = jnp.full((bsz, vp - vocab), _NEG_INF, jnp.float32)
        lp = jnp.concatenate([logits, pad], axis=1)
    else:
        lp = logits
    kb = jnp.broadcast_to(
        jnp.clip(k.astype(jnp.int32), 1, vocab)[:, None], (bsz, _LANE))
    pb = jnp.broadcast_to(p.astype(jnp.float32)[:, None], (bsz, _LANE))
    out = pl.pallas_call(
        _body,
        grid=(bsz // rblk,),
        in_specs=[
            pl.BlockSpec((rblk, vp), lambda i: (i, 0)),
            pl.BlockSpec((rblk, _LANE), lambda i: (i, 0)),
            pl.BlockSpec((rblk, _LANE), lambda i: (i, 0)),
        ],
        out_specs=pl.BlockSpec((rblk, vp), lambda i: (i, 0)),
        out_shape=jax.ShapeDtypeStruct((bsz, vp), jnp.float32),
        scratch_shapes=[
            pltpu.VMEM((rblk, vp), jnp.int32),
            pltpu.VMEM((rblk, vp), jnp.float32),
        ],
    )(lp, kb, pb)
    return out[:, :vocab]


# speedup vs baseline: 95.7759x; 1.9411x over previous
import functools

import jax
import jax.numpy as jnp
from jax import lax
from jax.experimental import pallas as pl
from jax.experimental.pallas import tpu as pltpu

_LANE = 128
_INT_MIN = -2147483648
_MASK31 = 0x7FFFFFFF
_NEG_INF = float("-inf")


def _body(v_ref, k_ref, p_ref, o_ref, skey_ref, e_ref):
    v = v_ref[...]
    b = lax.bitcast_convert_type(v, jnp.int32)
    skey = jnp.where(b >= 0, b, b ^ _MASK31)
    skey_ref[...] = skey
    m = jnp.max(v, axis=1, keepdims=True)
    maxkey = jnp.max(skey, axis=1, keepdims=True)
    e = jnp.exp(v - m)
    e_ref[...] = e
    kv = k_ref[:, :1]
    pv = p_ref[:, :1]

    vp = v.shape[1]
    csz = 98 * 128

    def _rowsum(x):
        parts = [
            jnp.sum(x[:, j:min(j + csz, vp)], axis=1, keepdims=True)
            for j in range(0, vp, csz)
        ]
        tot = parts[0]
        for q in parts[1:]:
            tot = tot + q
        return tot

    def cnt_ge(cand):
        return _rowsum((skey_ref[...] >= cand).astype(jnp.int32))

    zero = jnp.zeros_like(kv)
    base = jnp.where(cnt_ge(zero) >= kv, 0, _INT_MIN)

    def step1(i, rem):
        bit = jnp.left_shift(jnp.int32(1), 30 - i)
        cand = base + (rem | bit)
        return jnp.where(cnt_ge(cand) >= kv, rem | bit, rem)

    tk = base + lax.fori_loop(0, 31, step1, zero)

    s = _rowsum(jnp.where(skey_ref[...] >= tk, e_ref[...], 0.0))
    ps = pv * s

    def mass_gt(cand):
        return _rowsum(jnp.where(skey_ref[...] > cand, e_ref[...], 0.0))

    base2 = jnp.where(mass_gt(zero) >= ps, 0, _INT_MIN)

    def step2(i, rem):
        bit = jnp.left_shift(jnp.int32(1), 30 - i)
        cand = base2 + (rem | bit)
        return jnp.where(mass_gt(cand) >= ps, rem | bit, rem)

    mp = base2 + lax.fori_loop(0, 31, step2, zero)

    sk = skey_ref[...]
    keep = (sk >= tk) & ((sk > mp) | (sk == maxkey))
    o_ref[...] = jnp.where(keep, v, _NEG_INF)


@functools.partial(jax.jit, static_argnames=())
def kernel(logits, k, p):
    bsz, vocab = logits.shape
    vp = pl.cdiv(vocab, _LANE) * _LANE
    rblk = 16
    logits = logits.astype(jnp.float32)
    if vp != vocab:
        pad = jnp.full((bsz, vp - vocab), _NEG_INF, jnp.float32)
        lp = jnp.concatenate([logits, pad], axis=1)
    else:
        lp = logits
    kb = jnp.broadcast_to(
        jnp.clip(k.astype(jnp.int32), 1, vocab)[:, None], (bsz, _LANE))
    pb = jnp.broadcast_to(p.astype(jnp.float32)[:, None], (bsz, _LANE))
    out = pl.pallas_call(
        _body,
        grid=(bsz // rblk,),
        in_specs=[
            pl.BlockSpec((rblk, vp), lambda i: (i, 0)),
            pl.BlockSpec((rblk, _LANE), lambda i: (i, 0)),
            pl.BlockSpec((rblk, _LANE), lambda i: (i, 0)),
        ],
        out_specs=pl.BlockSpec((rblk, vp), lambda i: (i, 0)),
        out_shape=jax.ShapeDtypeStruct((bsz, vp), jnp.float32),
        scratch_shapes=[
            pltpu.VMEM((rblk, vp), jnp.int32),
            pltpu.VMEM((rblk, vp), jnp.float32),
        ],
    )(lp, kb, pb)
    return out[:, :vocab]


# speedup vs baseline: 97.4557x; 1.0175x over previous
import functools

import jax
import jax.numpy as jnp
from jax import lax
from jax.experimental import pallas as pl
from jax.experimental.pallas import tpu as pltpu

_LANE = 128
_INT_MIN = -2147483648
_MASK31 = 0x7FFFFFFF
_NEG_INF = float("-inf")


def _body(v_ref, k_ref, p_ref, o_ref, skey_ref, e_ref):
    v = v_ref[...]
    b = lax.bitcast_convert_type(v, jnp.int32)
    skey = jnp.where(b >= 0, b, b ^ _MASK31)
    skey_ref[...] = skey
    m = jnp.max(v, axis=1, keepdims=True)
    maxkey = jnp.max(skey, axis=1, keepdims=True)
    e = jnp.exp(v - m)
    e_ref[...] = e
    kv = k_ref[:, :1]
    pv = p_ref[:, :1]

    vp = v.shape[1]
    csz = 98 * 128

    def _rowsum(x):
        parts = [
            jnp.sum(x[:, j:min(j + csz, vp)], axis=1, keepdims=True)
            for j in range(0, vp, csz)
        ]
        tot = parts[0]
        for q in parts[1:]:
            tot = tot + q
        return tot

    def cnt_ge(cand):
        return _rowsum((skey_ref[...] >= cand).astype(jnp.int32))

    zero = jnp.zeros_like(kv)
    base = jnp.where(cnt_ge(zero) >= kv, 0, _INT_MIN)

    def step1(i, rem):
        bit = jnp.left_shift(jnp.int32(1), 30 - i)
        cand = base + (rem | bit)
        return jnp.where(cnt_ge(cand) >= kv, rem | bit, rem)

    tk = base + lax.fori_loop(0, 31, step1, zero)

    s = _rowsum(jnp.where(skey_ref[...] >= tk, e_ref[...], 0.0))
    ps = pv * s

    def mass_gt(cand):
        ev = e_ref[...]
        eb = lax.bitcast_convert_type(ev, jnp.int32)
        return _rowsum(jnp.where(eb > cand, ev, 0.0))

    def step2(i, rem):
        bit = jnp.left_shift(jnp.int32(1), 29 - i)
        cand = rem | bit
        return jnp.where(mass_gt(cand) >= ps, rem | bit, rem)

    mp = lax.fori_loop(0, 30, step2, zero)

    sk = skey_ref[...]
    eb = lax.bitcast_convert_type(e_ref[...], jnp.int32)
    keep = (sk >= tk) & ((eb > mp) | (sk == maxkey))
    o_ref[...] = jnp.where(keep, v, _NEG_INF)


@functools.partial(jax.jit, static_argnames=())
def kernel(logits, k, p):
    bsz, vocab = logits.shape
    vp = pl.cdiv(vocab, _LANE) * _LANE
    rblk = 16
    logits = logits.astype(jnp.float32)
    if vp != vocab:
        pad = jnp.full((bsz, vp - vocab), _NEG_INF, jnp.float32)
        lp = jnp.concatenate([logits, pad], axis=1)
    else:
        lp = logits
    kb = jnp.broadcast_to(
        jnp.clip(k.astype(jnp.int32), 1, vocab)[:, None], (bsz, _LANE))
    pb = jnp.broadcast_to(p.astype(jnp.float32)[:, None], (bsz, _LANE))
    out = pl.pallas_call(
        _body,
        grid=(bsz // rblk,),
        in_specs=[
            pl.BlockSpec((rblk, vp), lambda i: (i, 0)),
            pl.BlockSpec((rblk, _LANE), lambda i: (i, 0)),
            pl.BlockSpec((rblk, _LANE), lambda i: (i, 0)),
        ],
        out_specs=pl.BlockSpec((rblk, vp), lambda i: (i, 0)),
        out_shape=jax.ShapeDtypeStruct((bsz, vp), jnp.float32),
        scratch_shapes=[
            pltpu.VMEM((rblk, vp), jnp.int32),
            pltpu.VMEM((rblk, vp), jnp.float32),
        ],
    )(lp, kb, pb)
    return out[:, :vocab]
